# baseline (speedup 1.0000x reference)
import functools

import jax
import jax.numpy as jnp
from jax import lax
from jax.experimental import pallas as pl
from jax.experimental.pallas import tpu as pltpu
from jax.experimental.pallas import tpu_sc as plsc

NN = 10000
NE = 320000
NE2 = NE + NN
NPAD = 10016
EPAD = NE2 + 2048
RT = 313
CAP = 12800
GB = 64
CAP4 = 12800
EC4 = 13312 + 16
EPAD4 = NE + 2048
NP24 = 10024


def _sload(ref, i):
    return ref[pl.ds(i, 16)][0]


def _dense(x, w, b):
    n, d = x.shape
    k = w.shape[1]
    blk = 2000

    def body(x_ref, w_ref, b_ref, o_ref):
        o_ref[...] = jnp.dot(x_ref[...], w_ref[...],
                             preferred_element_type=jnp.float32) + b_ref[...]

    return pl.pallas_call(
        body,
        grid=(n // blk,),
        in_specs=[
            pl.BlockSpec((blk, d), lambda i: (i, 0)),
            pl.BlockSpec((d, k), lambda i: (0, 0)),
            pl.BlockSpec((k,), lambda i: (0,)),
        ],
        out_specs=pl.BlockSpec((blk, k), lambda i: (i, 0)),
        out_shape=jax.ShapeDtypeStruct((n, k), x.dtype),
    )(x, w, b)


def _tc_layer(h, wc, cs2, cd2):
    n, din = h.shape
    blk = 2000

    def body(h_ref, w_ref, cs_ref, cd_ref, hw_ref, as_ref, ad_ref, mm_ref):
        i = pl.program_id(0)
        hw = jnp.dot(h_ref[...], w_ref[...], preferred_element_type=jnp.float32)
        hw_ref[...] = jnp.concatenate(
            [hw, jnp.zeros((blk, 64), jnp.float32)], axis=1)
        av = jnp.dot(hw, cs_ref[...], preferred_element_type=jnp.float32)
        dv = jnp.dot(hw, cd_ref[...], preferred_element_type=jnp.float32)
        as_ref[...] = av
        ad_ref[...] = dv

        @pl.when(i == 0)
        def _():
            mm_ref[...] = jnp.full((1, 2), -1e30, jnp.float32)
        cur = mm_ref[...]
        new = jnp.stack([jnp.max(av), jnp.max(dv)])[None, :]
        mm_ref[...] = jnp.maximum(cur, new)

    return pl.pallas_call(
        body,
        grid=(n // blk,),
        in_specs=[
            pl.BlockSpec((blk, din), lambda i: (i, 0)),
            pl.BlockSpec((din, 64), lambda i: (0, 0)),
            pl.BlockSpec((64, 1), lambda i: (0, 0)),
            pl.BlockSpec((64, 1), lambda i: (0, 0)),
        ],
        out_specs=[
            pl.BlockSpec((blk, 128), lambda i: (i, 0)),
            pl.BlockSpec((blk, 1), lambda i: (i, 0)),
            pl.BlockSpec((blk, 1), lambda i: (i, 0)),
            pl.BlockSpec((1, 2), lambda i: (0, 0)),
        ],
        out_shape=[
            jax.ShapeDtypeStruct((n, 128), jnp.float32),
            jax.ShapeDtypeStruct((n, 1), jnp.float32),
            jax.ShapeDtypeStruct((n, 1), jnp.float32),
            jax.ShapeDtypeStruct((1, 2), jnp.float32),
        ],
    )(h, wc, cs2, cd2)


def _make_gat_sc(sigmoid):
    mesh = plsc.VectorSubcoreMesh(core_axis_name="c", subcore_axis_name="s")

    @functools.partial(
        pl.kernel,
        out_type=jax.ShapeDtypeStruct((NPAD * 64,), jnp.float32),
        mesh=mesh,
        compiler_params=pltpu.CompilerParams(needs_layout_passes=False),
        scratch_types=[
            pltpu.VMEM((NN,), jnp.float32),
            pltpu.VMEM((NN,), jnp.float32),
            pltpu.VMEM((48,), jnp.int32),
            pltpu.VMEM((64,), jnp.float32),
            pltpu.VMEM((16,), jnp.float32),
            pltpu.VMEM((1024,), jnp.int32),
            pltpu.VMEM((1024,), jnp.int32),
            pltpu.VMEM((CAP,), jnp.int32),
            pltpu.VMEM((CAP + 16,), jnp.int32),
            pltpu.VMEM((CAP + 16,), jnp.float32),
            pltpu.VMEM((336,), jnp.float32),
            pltpu.VMEM(((RT + 1) * 64,), jnp.float32),
            pltpu.VMEM((2, GB, 128), jnp.float32),
            pltpu.SemaphoreType.DMA,
            pltpu.SemaphoreType.DMA,
        ],
    )
    def k(hw_h, asv_h, adv_h, srcd_h, dstd_h, offs_h, bc_h, mx_h, hn_h,
          asv_v, adv_v, offs_v, bc_v, mx_v, sstage, dstage, smy, dmy, ev,
          sloc, outl, ring, sem0, sem1):
        wid = lax.axis_index("s") * 2 + lax.axis_index("c")
        lo = wid * RT
        hi = jnp.minimum(lo + RT, NN)
        pltpu.sync_copy(asv_h, asv_v)
        pltpu.sync_copy(adv_h, adv_v)
        pltpu.sync_copy(offs_h, offs_v)
        pltpu.sync_copy(bc_h, bc_v)
        pltpu.sync_copy(mx_h, mx_v)
        e_lo = _sload(offs_v, wid)
        e_hi = _sload(offs_v, wid + 1)
        e_lo8 = (e_lo // 8) * 8
        cnt = jnp.minimum(e_hi - e_lo8, CAP)
        mx = mx_v[pl.ds(0, 16)]

        z16 = jnp.zeros((16,), jnp.float32)

        def z1(g, _):
            sloc[pl.ds(g * 16, 16)] = z16
            return 0
        lax.fori_loop(0, 21, z1, 0)

        def z2(i, _):
            outl[pl.ds(i * 16, 16)] = z16
            return 0
        lax.fori_loop(0, (RT + 1) * 4, z2, 0)

        iota = lax.iota(jnp.int32, 16)
        nch = (cnt + 1023) // 1024

        def ch(c, _):
            pltpu.sync_copy(srcd_h.at[pl.ds(e_lo8 + c * 1024, 1024)], sstage)
            pltpu.sync_copy(dstd_h.at[pl.ds(e_lo8 + c * 1024, 1024)], dstage)

            def grp(g, _):
                kk = c * 1024 + g * 16
                s16 = sstage[pl.ds(g * 16, 16)]
                d16 = dstage[pl.ds(g * 16, 16)]
                a = (plsc.load_gather(asv_v, [s16])
                     + plsc.load_gather(adv_v, [d16]))
                a = jnp.where(a > 0, a, 0.2 * a)
                e = jnp.exp(a - mx)
                ok = ((kk + iota) < cnt) & (d16 >= lo) & (d16 < hi)
                dl = jnp.where(ok, d16 - lo, RT)
                plsc.addupdate_scatter(sloc, [dl], e)
                smy[pl.ds(kk, 16)] = s16
                dmy[pl.ds(kk, 16)] = dl
                ev[pl.ds(kk, 16)] = e
                return 0
            lax.fori_loop(0, 64, grp, 0)
            return 0
        lax.fori_loop(0, nch, ch, 0)

        def inv(g, _):
            sloc[pl.ds(g * 16, 16)] = 1.0 / (sloc[pl.ds(g * 16, 16)] + 1e-16)
            return 0
        lax.fori_loop(0, 20, inv, 0)

        nb = (cnt + GB - 1) // GB

        def _start(b):
            idx = smy.at[pl.ds(b * GB, GB)]

            @pl.when(lax.rem(b, 2) == 0)
            def _():
                pltpu.make_async_copy(hw_h.at[idx], ring.at[0], sem0).start()

            @pl.when(lax.rem(b, 2) == 1)
            def _():
                pltpu.make_async_copy(hw_h.at[idx], ring.at[1], sem1).start()

        def _wait(b):
            idx = smy.at[pl.ds(b * GB, GB)]

            @pl.when(lax.rem(b, 2) == 0)
            def _():
                pltpu.make_async_copy(hw_h.at[idx], ring.at[0], sem0).wait()

            @pl.when(lax.rem(b, 2) == 1)
            def _():
                pltpu.make_async_copy(hw_h.at[idx], ring.at[1], sem1).wait()

        @pl.when(nb > 0)
        def _():
            _start(0)

        def p2(b, _):
            _wait(b)

            @pl.when(b + 1 < nb)
            def _():
                _start(b + 1)
            slot = lax.rem(b, 2)

            def edge(j, _):
                kk = b * GB + j
                d = _sload(dmy, kk)
                al = _sload(ev, kk)
                for jj in range(4):
                    plsc.addupdate(outl.at[pl.ds(d * 64 + jj * 16, 16)],
                                   al * ring[slot, j, pl.ds(jj * 16, 16)])
                return 0
            lax.fori_loop(0, GB, edge, 0)
            return 0
        lax.fori_loop(0, nb, p2, 0)

        def fin(d, _):
            iv = _sload(sloc, d)
            for j in range(4):
                sl_ = pl.ds(d * 64 + j * 16, 16)
                v = outl[sl_] * iv + bc_v[pl.ds(j * 16, 16)]
                if sigmoid:
                    v = 1.0 / (1.0 + jnp.exp(-v))
                else:
                    v = jnp.maximum(v, 0.0)
                outl[sl_] = v
            return 0
        lax.fori_loop(0, RT, fin, 0)
        pltpu.sync_copy(outl.at[pl.ds(0, RT * 64)],
                        hn_h.at[pl.ds(lo * 64, RT * 64)])

    return k


_gat_sc_relu = _make_gat_sc(False)
_gat_sc_sig = _make_gat_sc(True)


def _edge_mlp_tc(h, wma, wmb, bm1):
    n = h.shape[0]
    blk = 2000

    def body(h_ref, wa_ref, wb_ref, b_ref, p_ref, q_ref):
        hh = h_ref[...]
        p_ref[...] = jnp.dot(hh, wa_ref[...],
                             preferred_element_type=jnp.float32) + b_ref[...]
        q = jnp.dot(hh, wb_ref[...], preferred_element_type=jnp.float32)
        q_ref[...] = jnp.concatenate(
            [q, jnp.zeros((blk, 96), jnp.float32)], axis=1)

    return pl.pallas_call(
        body,
        grid=(n // blk,),
        in_specs=[
            pl.BlockSpec((blk, 64), lambda i: (i, 0)),
            pl.BlockSpec((64, 32), lambda i: (0, 0)),
            pl.BlockSpec((64, 32), lambda i: (0, 0)),
            pl.BlockSpec((32,), lambda i: (0,)),
        ],
        out_specs=[
            pl.BlockSpec((blk, 32), lambda i: (i, 0)),
            pl.BlockSpec((blk, 128), lambda i: (i, 0)),
        ],
        out_shape=[
            jax.ShapeDtypeStruct((n, 32), jnp.float32),
            jax.ShapeDtypeStruct((n, 128), jnp.float32),
        ],
    )(h, wma, wmb, bm1)


def _make_out_sc():
    mesh = plsc.VectorSubcoreMesh(core_axis_name="c", subcore_axis_name="s")

    @functools.partial(
        pl.kernel,
        out_type=jax.ShapeDtypeStruct((NN * NN,), jnp.float32),
        mesh=mesh,
        compiler_params=pltpu.CompilerParams(needs_layout_passes=False),
        scratch_types=[
            pltpu.VMEM((336,), jnp.int32),
            pltpu.VMEM((328, 32), jnp.float32),
            pltpu.VMEM((EC4,), jnp.int32),
            pltpu.VMEM((EC4,), jnp.float32),
            pltpu.VMEM((1040,), jnp.int32),
            pltpu.VMEM((1024,), jnp.int32),
            pltpu.VMEM((48,), jnp.float32),
            pltpu.VMEM((2, GB, 128), jnp.float32),
            pltpu.VMEM((NN,), jnp.float32),
            pltpu.VMEM((NN,), jnp.float32),
            pltpu.VMEM((NN,), jnp.float32),
            pltpu.VMEM((NN,), jnp.float32),
            pltpu.SemaphoreType.DMA,
            pltpu.SemaphoreType.DMA,
            pltpu.SemaphoreType.DMA,
            pltpu.SemaphoreType.DMA,
            pltpu.SemaphoreType.DMA,
            pltpu.SemaphoreType.DMA,
        ],
    )
    def k(p_h, q_h, srcs_h, dsts_h, roff_h, w2_h, out_h,
          poff_v, pmy, dmy, ev, sstage, dstage, w2v, qring,
          rb0, rb1, rb2, rb3,
          gsem0, gsem1, rsem0, rsem1, rsem2, rsem3):
        rbs = (rb0, rb1, rb2, rb3)
        wid = lax.axis_index("s") * 2 + lax.axis_index("c")
        lo = wid * RT
        hi = jnp.minimum(lo + RT, NN)
        rmy = hi - lo
        lo8 = (lo // 8) * 8
        skew = lo - lo8
        pltpu.sync_copy(roff_h.at[pl.ds(lo8, 336)], poff_v)
        pltpu.sync_copy(p_h.at[pl.ds(lo8, 328)], pmy)
        pltpu.sync_copy(w2_h, w2v)
        er_lo = _sload(poff_v, skew)
        er_hi = _sload(poff_v, skew + rmy)
        er8 = (er_lo // 8) * 8
        cnt = jnp.minimum(er_hi - er8, CAP4)
        w2a = w2v[pl.ds(0, 16)]
        w2b = w2v[pl.ds(16, 16)]
        bm2s = _sload(w2v, 32)
        iota = lax.iota(jnp.int32, 16)
        z16 = jnp.zeros((16,), jnp.float32)

        def zb(g, _):
            for s in range(4):
                rbs[s][pl.ds(g * 16, 16)] = z16
            return 0
        lax.fori_loop(0, NN // 16, zb, 0)

        nch = (cnt + 1023) // 1024

        def qstart(b, dst_ref):
            idx = dst_ref.at[pl.ds(b * GB, GB)]

            @pl.when(lax.rem(b, 2) == 0)
            def _():
                pltpu.make_async_copy(q_h.at[idx], qring.at[0], gsem0).start()

            @pl.when(lax.rem(b, 2) == 1)
            def _():
                pltpu.make_async_copy(q_h.at[idx], qring.at[1], gsem1).start()

        def qwait(b, dst_ref):
            idx = dst_ref.at[pl.ds(b * GB, GB)]

            @pl.when(lax.rem(b, 2) == 0)
            def _():
                pltpu.make_async_copy(q_h.at[idx], qring.at[0], gsem0).wait()

            @pl.when(lax.rem(b, 2) == 1)
            def _():
                pltpu.make_async_copy(q_h.at[idx], qring.at[1], gsem1).wait()

        def ch(c, _):
            base = er8 + c * 1024
            pltpu.sync_copy(srcs_h.at[pl.ds(base, 1024)],
                            sstage.at[pl.ds(0, 1024)])
            pltpu.sync_copy(dsts_h.at[pl.ds(base, 1024)], dstage)

            def cp(g, _):
                dmy[pl.ds(c * 1024 + g * 16, 16)] = dstage[pl.ds(g * 16, 16)]
                return 0
            lax.fori_loop(0, 64, cp, 0)
            qstart(0, dstage)

            def bat(b, _):
                qwait(b, dstage)

                @pl.when(b + 1 < 1024 // GB)
                def _():
                    qstart(b + 1, dstage)
                slot = lax.rem(b, 2)

                def edge(j, _):
                    kj = b * GB + j
                    kk = c * 1024 + kj
                    sl = jnp.clip(_sload(sstage, kj) - lo8, 0, 327)
                    v0 = jnp.maximum(
                        pmy[sl, pl.ds(0, 16)] + qring[slot, j, pl.ds(0, 16)],
                        0.0)
                    v1 = jnp.maximum(
                        pmy[sl, pl.ds(16, 16)] + qring[slot, j, pl.ds(16, 16)],
                        0.0)
                    t = v0 * w2a + v1 * w2b
                    e = jnp.sum(t) + bm2s
                    plsc.store_scatter(ev, [jnp.full((16,), kk, jnp.int32)],
                                       jnp.full((16,), e, jnp.float32),
                                       mask=iota == 0)
                    return 0
                lax.fori_loop(0, GB, edge, 0)
                return 0
            lax.fori_loop(0, 1024 // GB, bat, 0)
            return 0
        lax.fori_loop(0, nch, ch, 0)

        rsems = (rsem0, rsem1, rsem2, rsem3)

        def row(d, _):
            r = lo + d
            ri = skew + d
            k0 = _sload(poff_v, ri) - er8
            k1 = jnp.minimum(_sload(poff_v, ri + 1) - er8, cnt)
            for s in range(4):
                @pl.when(lax.rem(d, 4) == s)
                def _(s=s, r=r, d=d, ri=ri, k0=k0, k1=k1):
                    sem = rsems[s]
                    rb = rbs[s]

                    @pl.when(d >= 4)
                    def _():
                        pltpu.make_async_copy(
                            rb, out_h.at[pl.ds((r - 4) * NN, NN)], sem).wait()
                        p0 = _sload(poff_v, ri - 4) - er8
                        p1 = jnp.minimum(_sload(poff_v, ri - 3) - er8, cnt)

                        def zg(g, _):
                            i16 = p0 + g * 16 + iota
                            msk = i16 < p1
                            d16 = plsc.load_gather(dmy, [i16])
                            plsc.store_scatter(rb, [d16], z16, mask=msk)
                            return 0
                        lax.fori_loop(0, (p1 - p0 + 15) // 16, zg, 0)

                    def sg(g, _):
                        i16 = k0 + g * 16 + iota
                        msk = i16 < k1
                        d16 = plsc.load_gather(dmy, [i16])
                        e16 = plsc.load_gather(ev, [i16])
                        plsc.addupdate_scatter(rb, [d16], e16, mask=msk)
                        return 0
                    lax.fori_loop(0, (k1 - k0 + 15) // 16, sg, 0)
                    pltpu.make_async_copy(
                        rb, out_h.at[pl.ds(r * NN, NN)], sem).start()
            return 0
        lax.fori_loop(0, rmy, row, 0)

        for jx in range(4):
            rr = rmy - 4 + jx

            @pl.when(rr >= 0)
            def _(rr=rr):
                for s in range(4):
                    @pl.when(lax.rem(rr, 4) == s)
                    def _(s=s, rr=rr):
                        pltpu.make_async_copy(
                            rbs[s],
                            out_h.at[pl.ds((lo + rr) * NN, NN)],
                            rsems[s]).wait()

    return k


_out_sc = _make_out_sc()


def kernel(x, edge_index, W1, b1, Wc1, as1, ad1, bc1, Wc2, as2, ad2, bc2,
           Wc3, as3, ad3, bc3, Wc4, as4, ad4, bc4, Wc5, as5, ad5, bc5,
           Wm1, bm1, Wm2, bm2):
    src, dst = edge_index[0], edge_index[1]
    loop = jnp.arange(NN, dtype=src.dtype)
    src2 = jnp.concatenate([src, loop])
    dst2 = jnp.concatenate([dst, loop])
    perm_d = jnp.argsort(dst2)
    srcd = src2[perm_d]
    dstd = dst2[perm_d]
    bounds = jnp.minimum(jnp.arange(33, dtype=jnp.int32) * RT, NN)
    offs_d = jnp.searchsorted(dstd, bounds).astype(jnp.int32)
    offs_d = jnp.pad(offs_d, (0, 15))
    srcd_p = jnp.pad(srcd, (0, EPAD - NE2))
    dstd_p = jnp.pad(dstd, (0, EPAD - NE2))

    h = _dense(x, W1, b1)
    layers = [
        (Wc1, as1, ad1, bc1, False), (Wc2, as2, ad2, bc2, False),
        (Wc3, as3, ad3, bc3, False), (Wc4, as4, ad4, bc4, False),
        (Wc5, as5, ad5, bc5, True),
    ]
    for wc, avs, avd, bc, sig in layers:
        hw, asv, adv, mm = _tc_layer(h, wc, avs[:, None], avd[:, None])
        m = mm[0, 0] + mm[0, 1]
        mx = jnp.maximum(m, 0.2 * m)
        mxv = jnp.full((16,), mx, jnp.float32)
        fn = _gat_sc_sig if sig else _gat_sc_relu
        hn = fn(hw, asv[:, 0], adv[:, 0], srcd_p, dstd_p, offs_d, bc, mxv)
        h = hn.reshape(NPAD, 64)[:NN]

    perm_s = jnp.argsort(src)
    srcs = src[perm_s]
    dsts = dst[perm_s]
    row_offs = jnp.searchsorted(srcs, jnp.arange(NN + 1, dtype=jnp.int32))
    row_offs = jnp.pad(row_offs.astype(jnp.int32), (0, 10032 - (NN + 1)))
    srcs_p = jnp.pad(srcs, (0, EPAD4 - NE))
    dsts_p = jnp.pad(dsts, (0, EPAD4 - NE))
    P, Q128 = _edge_mlp_tc(h, Wm1[:64], Wm1[64:], bm1)
    P_p = jnp.pad(P, ((0, NP24 - NN), (0, 0)))
    w2pack = jnp.concatenate([Wm2[:, 0], jnp.full((16,), bm2[0], jnp.float32)])
    flat = _out_sc(P_p, Q128, srcs_p, dsts_p, row_offs, w2pack)
    return flat.reshape(NN, NN)


# speedup vs baseline: 2.0731x; 2.0731x over previous
import functools

import jax
import jax.numpy as jnp
from jax import lax
from jax.experimental import pallas as pl
from jax.experimental.pallas import tpu as pltpu
from jax.experimental.pallas import tpu_sc as plsc

NN = 10000
NE = 320000
NE2 = NE + NN
NPAD = 10016
EPAD = NE2 + 2048
RT = 313
CAP = 12800
GB = 64
CAP4 = 12800
EC4 = 13312 + 16
EPAD4 = NE + 2048
NP24 = 10024


def _sload(ref, i):
    return ref[pl.ds(i, 16)][0]


def _dense(x, w, b):
    n, d = x.shape
    k = w.shape[1]
    blk = 2000

    def body(x_ref, w_ref, b_ref, o_ref):
        o_ref[...] = jnp.dot(x_ref[...], w_ref[...],
                             preferred_element_type=jnp.float32) + b_ref[...]

    return pl.pallas_call(
        body,
        grid=(n // blk,),
        in_specs=[
            pl.BlockSpec((blk, d), lambda i: (i, 0)),
            pl.BlockSpec((d, k), lambda i: (0, 0)),
            pl.BlockSpec((k,), lambda i: (0,)),
        ],
        out_specs=pl.BlockSpec((blk, k), lambda i: (i, 0)),
        out_shape=jax.ShapeDtypeStruct((n, k), x.dtype),
    )(x, w, b)


def _tc_layer(h, wc, cs2, cd2):
    n, din = h.shape
    blk = 2000

    def body(h_ref, w_ref, cs_ref, cd_ref, hw_ref, as_ref, ad_ref, mm_ref):
        i = pl.program_id(0)
        hw = jnp.dot(h_ref[...], w_ref[...], preferred_element_type=jnp.float32)
        hw_ref[...] = jnp.concatenate(
            [hw, jnp.zeros((blk, 64), jnp.float32)], axis=1)
        av = jnp.dot(hw, cs_ref[...], preferred_element_type=jnp.float32)
        dv = jnp.dot(hw, cd_ref[...], preferred_element_type=jnp.float32)
        as_ref[...] = av
        ad_ref[...] = dv

        @pl.when(i == 0)
        def _():
            mm_ref[...] = jnp.full((1, 2), -1e30, jnp.float32)
        cur = mm_ref[...]
        new = jnp.stack([jnp.max(av), jnp.max(dv)])[None, :]
        mm_ref[...] = jnp.maximum(cur, new)

    return pl.pallas_call(
        body,
        grid=(n // blk,),
        in_specs=[
            pl.BlockSpec((blk, din), lambda i: (i, 0)),
            pl.BlockSpec((din, 64), lambda i: (0, 0)),
            pl.BlockSpec((64, 1), lambda i: (0, 0)),
            pl.BlockSpec((64, 1), lambda i: (0, 0)),
        ],
        out_specs=[
            pl.BlockSpec((blk, 128), lambda i: (i, 0)),
            pl.BlockSpec((blk, 1), lambda i: (i, 0)),
            pl.BlockSpec((blk, 1), lambda i: (i, 0)),
            pl.BlockSpec((1, 2), lambda i: (0, 0)),
        ],
        out_shape=[
            jax.ShapeDtypeStruct((n, 128), jnp.float32),
            jax.ShapeDtypeStruct((n, 1), jnp.float32),
            jax.ShapeDtypeStruct((n, 1), jnp.float32),
            jax.ShapeDtypeStruct((1, 2), jnp.float32),
        ],
    )(h, wc, cs2, cd2)


def _make_gat_sc(sigmoid):
    mesh = plsc.VectorSubcoreMesh(core_axis_name="c", subcore_axis_name="s")

    @functools.partial(
        pl.kernel,
        out_type=jax.ShapeDtypeStruct((NPAD * 64,), jnp.float32),
        mesh=mesh,
        compiler_params=pltpu.CompilerParams(needs_layout_passes=False),
        scratch_types=[
            pltpu.VMEM((NN,), jnp.float32),
            pltpu.VMEM((NN,), jnp.float32),
            pltpu.VMEM((48,), jnp.int32),
            pltpu.VMEM((64,), jnp.float32),
            pltpu.VMEM((16,), jnp.float32),
            pltpu.VMEM((1024,), jnp.int32),
            pltpu.VMEM((1024,), jnp.int32),
            pltpu.VMEM((CAP,), jnp.int32),
            pltpu.VMEM((CAP + 16,), jnp.int32),
            pltpu.VMEM((CAP + 16,), jnp.float32),
            pltpu.VMEM((336,), jnp.float32),
            pltpu.VMEM(((RT + 1) * 64,), jnp.float32),
            pltpu.VMEM((2, GB, 128), jnp.float32),
            pltpu.SemaphoreType.DMA,
            pltpu.SemaphoreType.DMA,
        ],
    )
    def k(hw_h, asv_h, adv_h, srcd_h, dstd_h, offs_h, bc_h, mx_h, hn_h,
          asv_v, adv_v, offs_v, bc_v, mx_v, sstage, dstage, smy, dmy, ev,
          sloc, outl, ring, sem0, sem1):
        wid = lax.axis_index("s") * 2 + lax.axis_index("c")
        lo = wid * RT
        hi = jnp.minimum(lo + RT, NN)
        pltpu.sync_copy(asv_h, asv_v)
        pltpu.sync_copy(adv_h, adv_v)
        pltpu.sync_copy(offs_h, offs_v)
        pltpu.sync_copy(bc_h, bc_v)
        pltpu.sync_copy(mx_h, mx_v)
        e_lo = _sload(offs_v, wid)
        e_hi = _sload(offs_v, wid + 1)
        e_lo8 = (e_lo // 8) * 8
        cnt = jnp.minimum(e_hi - e_lo8, CAP)
        mx = mx_v[pl.ds(0, 16)]

        z16 = jnp.zeros((16,), jnp.float32)

        def z1(g, _):
            sloc[pl.ds(g * 16, 16)] = z16
            return 0
        lax.fori_loop(0, 21, z1, 0)

        def z2(i, _):
            outl[pl.ds(i * 16, 16)] = z16
            return 0
        lax.fori_loop(0, (RT + 1) * 4, z2, 0)

        iota = lax.iota(jnp.int32, 16)
        nch = (cnt + 1023) // 1024

        def ch(c, _):
            pltpu.sync_copy(srcd_h.at[pl.ds(e_lo8 + c * 1024, 1024)], sstage)
            pltpu.sync_copy(dstd_h.at[pl.ds(e_lo8 + c * 1024, 1024)], dstage)

            def grp(g, _):
                kk = c * 1024 + g * 16
                s16 = sstage[pl.ds(g * 16, 16)]
                d16 = dstage[pl.ds(g * 16, 16)]
                a = (plsc.load_gather(asv_v, [s16])
                     + plsc.load_gather(adv_v, [d16]))
                a = jnp.where(a > 0, a, 0.2 * a)
                e = jnp.exp(a - mx)
                ok = ((kk + iota) < cnt) & (d16 >= lo) & (d16 < hi)
                dl = jnp.where(ok, d16 - lo, RT)
                plsc.addupdate_scatter(sloc, [dl], e)
                smy[pl.ds(kk, 16)] = s16
                dmy[pl.ds(kk, 16)] = dl
                ev[pl.ds(kk, 16)] = e
                return 0
            lax.fori_loop(0, 64, grp, 0)
            return 0
        lax.fori_loop(0, nch, ch, 0)

        def inv(g, _):
            sloc[pl.ds(g * 16, 16)] = 1.0 / (sloc[pl.ds(g * 16, 16)] + 1e-16)
            return 0
        lax.fori_loop(0, 20, inv, 0)

        nb = (cnt + GB - 1) // GB

        def _start(b):
            idx = smy.at[pl.ds(b * GB, GB)]

            @pl.when(lax.rem(b, 2) == 0)
            def _():
                pltpu.make_async_copy(hw_h.at[idx], ring.at[0], sem0).start()

            @pl.when(lax.rem(b, 2) == 1)
            def _():
                pltpu.make_async_copy(hw_h.at[idx], ring.at[1], sem1).start()

        def _wait(b):
            idx = smy.at[pl.ds(b * GB, GB)]

            @pl.when(lax.rem(b, 2) == 0)
            def _():
                pltpu.make_async_copy(hw_h.at[idx], ring.at[0], sem0).wait()

            @pl.when(lax.rem(b, 2) == 1)
            def _():
                pltpu.make_async_copy(hw_h.at[idx], ring.at[1], sem1).wait()

        @pl.when(nb > 0)
        def _():
            _start(0)

        def p2(b, _):
            _wait(b)

            @pl.when(b + 1 < nb)
            def _():
                _start(b + 1)
            slot = lax.rem(b, 2)

            def edge(j, _):
                kk = b * GB + j
                d = _sload(dmy, kk)
                al = _sload(ev, kk)
                for jj in range(4):
                    plsc.addupdate(outl.at[pl.ds(d * 64 + jj * 16, 16)],
                                   al * ring[slot, j, pl.ds(jj * 16, 16)])
                return 0
            lax.fori_loop(0, GB, edge, 0)
            return 0
        lax.fori_loop(0, nb, p2, 0)

        def fin(d, _):
            iv = _sload(sloc, d)
            for j in range(4):
                sl_ = pl.ds(d * 64 + j * 16, 16)
                v = outl[sl_] * iv + bc_v[pl.ds(j * 16, 16)]
                if sigmoid:
                    v = 1.0 / (1.0 + jnp.exp(-v))
                else:
                    v = jnp.maximum(v, 0.0)
                outl[sl_] = v
            return 0
        lax.fori_loop(0, RT, fin, 0)
        pltpu.sync_copy(outl.at[pl.ds(0, RT * 64)],
                        hn_h.at[pl.ds(lo * 64, RT * 64)])

    return k


_gat_sc_relu = _make_gat_sc(False)
_gat_sc_sig = _make_gat_sc(True)


def _edge_mlp_tc(h, wma, wmb, bm1):
    n = h.shape[0]
    blk = 2000

    def body(h_ref, wa_ref, wb_ref, b_ref, p_ref, q_ref):
        hh = h_ref[...]
        p_ref[...] = jnp.dot(hh, wa_ref[...],
                             preferred_element_type=jnp.float32) + b_ref[...]
        q = jnp.dot(hh, wb_ref[...], preferred_element_type=jnp.float32)
        q_ref[...] = jnp.concatenate(
            [q, jnp.zeros((blk, 96), jnp.float32)], axis=1)

    return pl.pallas_call(
        body,
        grid=(n // blk,),
        in_specs=[
            pl.BlockSpec((blk, 64), lambda i: (i, 0)),
            pl.BlockSpec((64, 32), lambda i: (0, 0)),
            pl.BlockSpec((64, 32), lambda i: (0, 0)),
            pl.BlockSpec((32,), lambda i: (0,)),
        ],
        out_specs=[
            pl.BlockSpec((blk, 32), lambda i: (i, 0)),
            pl.BlockSpec((blk, 128), lambda i: (i, 0)),
        ],
        out_shape=[
            jax.ShapeDtypeStruct((n, 32), jnp.float32),
            jax.ShapeDtypeStruct((n, 128), jnp.float32),
        ],
    )(h, wma, wmb, bm1)


def _make_out_sc():
    mesh = plsc.VectorSubcoreMesh(core_axis_name="c", subcore_axis_name="s")

    @functools.partial(
        pl.kernel,
        out_type=jax.ShapeDtypeStruct((NN * NN,), jnp.float32),
        mesh=mesh,
        compiler_params=pltpu.CompilerParams(needs_layout_passes=False),
        scratch_types=[
            pltpu.VMEM((336,), jnp.int32),
            pltpu.VMEM((328, 32), jnp.float32),
            pltpu.VMEM((EC4,), jnp.int32),
            pltpu.VMEM((EC4,), jnp.float32),
            pltpu.VMEM((1040,), jnp.int32),
            pltpu.VMEM((1024,), jnp.int32),
            pltpu.VMEM((48,), jnp.float32),
            pltpu.VMEM((2, GB, 128), jnp.float32),
            pltpu.VMEM((NN,), jnp.float32),
            pltpu.VMEM((NN,), jnp.float32),
            pltpu.VMEM((NN,), jnp.float32),
            pltpu.VMEM((NN,), jnp.float32),
            pltpu.SemaphoreType.DMA,
            pltpu.SemaphoreType.DMA,
            pltpu.SemaphoreType.DMA,
            pltpu.SemaphoreType.DMA,
            pltpu.SemaphoreType.DMA,
            pltpu.SemaphoreType.DMA,
        ],
    )
    def k(p_h, q_h, srcs_h, dsts_h, roff_h, w2_h, out_h,
          poff_v, pmy, dmy, ev, sstage, dstage, w2v, qring,
          rb0, rb1, rb2, rb3,
          gsem0, gsem1, rsem0, rsem1, rsem2, rsem3):
        rbs = (rb0, rb1, rb2, rb3)
        wid = lax.axis_index("s") * 2 + lax.axis_index("c")
        lo = wid * RT
        hi = jnp.minimum(lo + RT, NN)
        rmy = hi - lo
        lo8 = (lo // 8) * 8
        skew = lo - lo8
        pltpu.sync_copy(roff_h.at[pl.ds(lo8, 336)], poff_v)
        pltpu.sync_copy(p_h.at[pl.ds(lo8, 328)], pmy)
        pltpu.sync_copy(w2_h, w2v)
        er_lo = _sload(poff_v, skew)
        er_hi = _sload(poff_v, skew + rmy)
        er8 = (er_lo // 8) * 8
        cnt = jnp.minimum(er_hi - er8, CAP4)
        w2a = w2v[pl.ds(0, 16)]
        w2b = w2v[pl.ds(16, 16)]
        bm2s = _sload(w2v, 32)
        iota = lax.iota(jnp.int32, 16)
        z16 = jnp.zeros((16,), jnp.float32)

        def zb(g, _):
            for s in range(4):
                rbs[s][pl.ds(g * 16, 16)] = z16
            return 0
        lax.fori_loop(0, NN // 16, zb, 0)

        nch = (cnt + 1023) // 1024

        def qstart(b, dst_ref):
            idx = dst_ref.at[pl.ds(b * GB, GB)]

            @pl.when(lax.rem(b, 2) == 0)
            def _():
                pltpu.make_async_copy(q_h.at[idx], qring.at[0], gsem0).start()

            @pl.when(lax.rem(b, 2) == 1)
            def _():
                pltpu.make_async_copy(q_h.at[idx], qring.at[1], gsem1).start()

        def qwait(b, dst_ref):
            idx = dst_ref.at[pl.ds(b * GB, GB)]

            @pl.when(lax.rem(b, 2) == 0)
            def _():
                pltpu.make_async_copy(q_h.at[idx], qring.at[0], gsem0).wait()

            @pl.when(lax.rem(b, 2) == 1)
            def _():
                pltpu.make_async_copy(q_h.at[idx], qring.at[1], gsem1).wait()

        def ch(c, _):
            base = er8 + c * 1024
            pltpu.sync_copy(srcs_h.at[pl.ds(base, 1024)],
                            sstage.at[pl.ds(0, 1024)])
            pltpu.sync_copy(dsts_h.at[pl.ds(base, 1024)], dstage)

            def cp(g, _):
                dmy[pl.ds(c * 1024 + g * 16, 16)] = dstage[pl.ds(g * 16, 16)]
                return 0
            lax.fori_loop(0, 64, cp, 0)
            qstart(0, dstage)

            def bat(b, _):
                qwait(b, dstage)

                @pl.when(b + 1 < 1024 // GB)
                def _():
                    qstart(b + 1, dstage)
                slot = lax.rem(b, 2)

                def edge(j, _):
                    kj = b * GB + j
                    kk = c * 1024 + kj
                    sl = jnp.clip(_sload(sstage, kj) - lo8, 0, 327)
                    v0 = jnp.maximum(
                        pmy[sl, pl.ds(0, 16)] + qring[slot, j, pl.ds(0, 16)],
                        0.0)
                    v1 = jnp.maximum(
                        pmy[sl, pl.ds(16, 16)] + qring[slot, j, pl.ds(16, 16)],
                        0.0)
                    t = v0 * w2a + v1 * w2b
                    e = jnp.sum(t) + bm2s
                    plsc.store_scatter(ev, [jnp.full((16,), kk, jnp.int32)],
                                       jnp.full((16,), e, jnp.float32),
                                       mask=iota == 0)
                    return 0
                lax.fori_loop(0, GB, edge, 0)
                return 0
            lax.fori_loop(0, 1024 // GB, bat, 0)
            return 0
        lax.fori_loop(0, nch, ch, 0)

        rsems = (rsem0, rsem1, rsem2, rsem3)

        def row(d, _):
            r = lo + d
            ri = skew + d
            k0 = _sload(poff_v, ri) - er8
            k1 = jnp.minimum(_sload(poff_v, ri + 1) - er8, cnt)
            for s in range(4):
                @pl.when(lax.rem(d, 4) == s)
                def _(s=s, r=r, d=d, ri=ri, k0=k0, k1=k1):
                    sem = rsems[s]
                    rb = rbs[s]

                    @pl.when(d >= 4)
                    def _():
                        pltpu.make_async_copy(
                            rb, out_h.at[pl.ds((r - 4) * NN, NN)], sem).wait()
                        p0 = _sload(poff_v, ri - 4) - er8
                        p1 = jnp.minimum(_sload(poff_v, ri - 3) - er8, cnt)

                        def zg(g, _):
                            i16 = p0 + g * 16 + iota
                            msk = i16 < p1
                            d16 = plsc.load_gather(dmy, [i16])
                            plsc.store_scatter(rb, [d16], z16, mask=msk)
                            return 0
                        lax.fori_loop(0, (p1 - p0 + 15) // 16, zg, 0)

                    def sg(g, _):
                        i16 = k0 + g * 16 + iota
                        msk = i16 < k1
                        d16 = plsc.load_gather(dmy, [i16])
                        e16 = plsc.load_gather(ev, [i16])
                        plsc.addupdate_scatter(rb, [d16], e16, mask=msk)
                        return 0
                    lax.fori_loop(0, (k1 - k0 + 15) // 16, sg, 0)
                    pltpu.make_async_copy(
                        rb, out_h.at[pl.ds(r * NN, NN)], sem).start()
            return 0
        lax.fori_loop(0, rmy, row, 0)

        for jx in range(4):
            rr = rmy - 4 + jx

            @pl.when(rr >= 0)
            def _(rr=rr):
                for s in range(4):
                    @pl.when(lax.rem(rr, 4) == s)
                    def _(s=s, rr=rr):
                        pltpu.make_async_copy(
                            rbs[s],
                            out_h.at[pl.ds((lo + rr) * NN, NN)],
                            rsems[s]).wait()

    return k


_out_sc = _make_out_sc()


def kernel(x, edge_index, W1, b1, Wc1, as1, ad1, bc1, Wc2, as2, ad2, bc2,
           Wc3, as3, ad3, bc3, Wc4, as4, ad4, bc4, Wc5, as5, ad5, bc5,
           Wm1, bm1, Wm2, bm2):
    src, dst = edge_index[0], edge_index[1]
    loop = jnp.arange(NN, dtype=src.dtype)
    src2 = jnp.concatenate([src, loop])
    dst2 = jnp.concatenate([dst, loop])
    dstd, srcd = lax.sort((dst2, src2), num_keys=1, is_stable=False)
    bounds = jnp.minimum(jnp.arange(33, dtype=jnp.int32) * RT, NN)
    offs_d = jnp.searchsorted(dstd, bounds).astype(jnp.int32)
    offs_d = jnp.pad(offs_d, (0, 15))
    srcd_p = jnp.pad(srcd, (0, EPAD - NE2))
    dstd_p = jnp.pad(dstd, (0, EPAD - NE2))

    h = _dense(x, W1, b1)
    layers = [
        (Wc1, as1, ad1, bc1, False), (Wc2, as2, ad2, bc2, False),
        (Wc3, as3, ad3, bc3, False), (Wc4, as4, ad4, bc4, False),
        (Wc5, as5, ad5, bc5, True),
    ]
    for wc, avs, avd, bc, sig in layers:
        hw, asv, adv, mm = _tc_layer(h, wc, avs[:, None], avd[:, None])
        m = mm[0, 0] + mm[0, 1]
        mx = jnp.maximum(m, 0.2 * m)
        mxv = jnp.full((16,), mx, jnp.float32)
        fn = _gat_sc_sig if sig else _gat_sc_relu
        hn = fn(hw, asv[:, 0], adv[:, 0], srcd_p, dstd_p, offs_d, bc, mxv)
        h = hn.reshape(NPAD, 64)[:NN]

    srcs, dsts = lax.sort((src, dst), num_keys=1, is_stable=False)
    row_offs = jnp.searchsorted(srcs, jnp.arange(NN + 1, dtype=jnp.int32))
    row_offs = jnp.pad(row_offs.astype(jnp.int32), (0, 10032 - (NN + 1)))
    srcs_p = jnp.pad(srcs, (0, EPAD4 - NE))
    dsts_p = jnp.pad(dsts, (0, EPAD4 - NE))
    P, Q128 = _edge_mlp_tc(h, Wm1[:64], Wm1[64:], bm1)
    P_p = jnp.pad(P, ((0, NP24 - NN), (0, 0)))
    w2pack = jnp.concatenate([Wm2[:, 0], jnp.full((16,), bm2[0], jnp.float32)])
    flat = _out_sc(P_p, Q128, srcs_p, dsts_p, row_offs, w2pack)
    return flat.reshape(NN, NN)
